# baseline (speedup 1.0000x reference)
import functools

import numpy as np
import jax
import jax.numpy as jnp
from jax import lax
from jax.experimental import pallas as pl
from jax.experimental.pallas import tpu as pltpu
from jax.experimental.pallas import tpu_sc as plsc

_H, _W = 480, 640
_FX, _FY, _CX, _CY = 600.0, 600.0, 319.5, 239.5
_NF = 8
_N = 8192
_NSTRAT = 19
_NSURF = 8
_S = _NSTRAT + _NSURF
_MIN_DEPTH = 0.07
_DIST_BEHIND = 0.1
_SURF_OFF = 0.05

_RPW = _N // 32
_NGRP = _RPW // 16

_LIMS = np.linspace(0.0, 1.0, _NSTRAT + 1, dtype=np.float32)
_LOWER = [float(x) for x in _LIMS[:-1]]
_WIDTH = [float(b - a) for a, b in zip(_LIMS[:-1], _LIMS[1:])]

_SORT8 = [(0, 1), (2, 3), (4, 5), (6, 7),
          (0, 2), (1, 3), (4, 6), (5, 7),
          (1, 2), (5, 6),
          (0, 4), (1, 5), (2, 6), (3, 7),
          (2, 4), (3, 5),
          (1, 2), (3, 4), (5, 6)]


def _cmpx(v, i, j):
    a, b = v[i], v[j]
    if a is None and b is None:
        return
    if a is None:
        v[i], v[j] = b, None
        return
    if b is None:
        return
    v[i], v[j] = jnp.minimum(a, b), jnp.maximum(a, b)


def _sorted27(strat, surf):
    s = list(surf)
    for i, j in _SORT8:
        _cmpx(s, i, j)
    v = list(strat) + [None] * 5 + s[::-1]
    for d in (16, 8, 4, 2, 1):
        for i in range(32):
            if i % (2 * d) < d:
                _cmpx(v, i, i + d)
    out = v[:_S]
    assert all(x is not None for x in out)
    return out


def _body(depth_hbm, tbl_hbm, ib_hbm, ih_hbm, iw_hbm, sn_hbm, sf_hbm,
          oz_hbm, opc_hbm,
          ib_v, ih_v, iw_v, idx_a, idx_b, dep_v, tbl_v, sn_v, sf_v,
          oz_v, opc_v, sem):
    info = plsc.get_sparse_core_info()
    wid = lax.axis_index("s") * info.num_cores + lax.axis_index("c")
    base = wid * _RPW

    cps = [
        pltpu.async_copy(ib_hbm.at[pl.ds(base, _RPW)], ib_v, sem),
        pltpu.async_copy(ih_hbm.at[pl.ds(base, _RPW)], ih_v, sem),
        pltpu.async_copy(iw_hbm.at[pl.ds(base, _RPW)], iw_v, sem),
        pltpu.async_copy(sn_hbm.at[pl.ds(base, _RPW)], sn_v, sem),
        pltpu.async_copy(sf_hbm.at[pl.ds(base, _RPW)], sf_v, sem),
        pltpu.async_copy(tbl_hbm, tbl_v, sem),
    ]
    for cp in cps:
        cp.wait()

    def _flat_idx(idx_ref, half):
        @plsc.parallel_loop(0, 8, unroll=2)
        def b2(g2):
            off = half * 128 + g2 * 16
            b = ib_v[pl.ds(off, 16)]
            h = ih_v[pl.ds(off, 16)]
            w = iw_v[pl.ds(off, 16)]
            idx_ref[pl.ds(g2 * 16, 16)] = (b * _H + h) * _W + w

    _flat_idx(idx_a, 0)
    _flat_idx(idx_b, 1)
    ga = pltpu.async_copy(depth_hbm.at[idx_a], dep_v.at[pl.ds(0, 128)], sem)
    gb = pltpu.async_copy(depth_hbm.at[idx_b], dep_v.at[pl.ds(128, 128)], sem)
    ga.wait()
    gb.wait()

    iota = lax.iota(jnp.int32, 16)

    @plsc.parallel_loop(0, _NGRP, unroll=2)
    def grp(g):
        off = g * 16
        rows = off + iota
        b = ib_v[pl.ds(off, 16)]
        h = ih_v[pl.ds(off, 16)]
        w = iw_v[pl.ds(off, 16)]
        dx = (w.astype(jnp.float32) - _CX) * (1.0 / _FX)
        dy = (h.astype(jnp.float32) - _CY) * (1.0 / _FY)

        def col(k):
            return jnp.full((16,), k, jnp.int32)

        tk = [plsc.load_gather(tbl_v, [b, col(k // 4), col(k % 4)])
              for k in range(12)]
        dwx = tk[0] * dx + tk[1] * dy + tk[2]
        dwy = tk[4] * dx + tk[5] * dy + tk[6]
        dwz = tk[8] * dx + tk[9] * dy + tk[10]
        dirs = (dwx, dwy, dwz)
        orig = (tk[3], tk[7], tk[11])

        d = dep_v[pl.ds(off, 16)]
        maxd = d + jnp.sign(d + 1e-8) * _DIST_BEHIND
        rng = maxd - _MIN_DEPTH

        strat = []
        for s in range(_NSTRAT):
            nz = plsc.load_gather(sn_v, [rows, col(s)])
            t = _LOWER[s] + _WIDTH[s] * nz
            strat.append(_MIN_DEPTH + t * rng)
        surf = [d]
        for u in range(1, _NSURF):
            nz = plsc.load_gather(sf_v, [rows, col(u)])
            surf.append(d + nz * _SURF_OFF)

        zs = _sorted27(strat, surf)

        for s in range(_S):
            oz_v[s, pl.ds(off, 16)] = zs[s]
            for c in range(3):
                opc_v[c, s, pl.ds(off, 16)] = orig[c] + dirs[c] * zs[s]

    oa = pltpu.async_copy(oz_v, oz_hbm.at[:, pl.ds(base, _RPW)], sem)
    ob = pltpu.async_copy(opc_v, opc_hbm.at[:, :, pl.ds(base, _RPW)], sem)
    oa.wait()
    ob.wait()


_sens = functools.partial(
    pl.kernel,
    mesh=plsc.VectorSubcoreMesh(core_axis_name="c", subcore_axis_name="s"),
    out_type=[
        jax.ShapeDtypeStruct((_S, _N), jnp.float32),
        jax.ShapeDtypeStruct((3, _S, _N), jnp.float32),
    ],
    scratch_types=[
        pltpu.VMEM((_RPW,), jnp.int32),
        pltpu.VMEM((_RPW,), jnp.int32),
        pltpu.VMEM((_RPW,), jnp.int32),
        pltpu.VMEM((128,), jnp.int32),
        pltpu.VMEM((128,), jnp.int32),
        pltpu.VMEM((_RPW,), jnp.float32),
        pltpu.VMEM((_NF, 4, 4), jnp.float32),
        pltpu.VMEM((_RPW, _NSTRAT), jnp.float32),
        pltpu.VMEM((_RPW, _NSURF), jnp.float32),
        pltpu.VMEM((_S, _RPW), jnp.float32),
        pltpu.VMEM((3, _S, _RPW), jnp.float32),
        pltpu.SemaphoreType.DMA,
    ],
    compiler_params=pltpu.CompilerParams(
        needs_layout_passes=False,
        disable_bounds_checks=True,
        disable_semaphore_checks=True,
    ),
)(_body)


def kernel(depth_batch, T_WC_batch, indices_b, indices_h, indices_w,
           strat_noise, surf_noise):
    depth_flat = depth_batch.reshape(_NF * _H * _W)
    zt, pct = _sens(depth_flat, T_WC_batch,
                    indices_b.astype(jnp.int32),
                    indices_h.astype(jnp.int32),
                    indices_w.astype(jnp.int32),
                    strat_noise, surf_noise)
    return jnp.transpose(pct, (2, 1, 0)), zt.T


# speedup vs baseline: 1.0356x; 1.0356x over previous
import functools

import numpy as np
import jax
import jax.numpy as jnp
from jax import lax
from jax.experimental import pallas as pl
from jax.experimental.pallas import tpu as pltpu
from jax.experimental.pallas import tpu_sc as plsc

_H, _W = 480, 640
_FX, _FY, _CX, _CY = 600.0, 600.0, 319.5, 239.5
_NF = 8
_N = 8192
_NSTRAT = 19
_NSURF = 8
_S = _NSTRAT + _NSURF
_MIN_DEPTH = 0.07
_DIST_BEHIND = 0.1
_SURF_OFF = 0.05

_RPW = _N // 32
_NGRP = _RPW // 16

_LIMS = np.linspace(0.0, 1.0, _NSTRAT + 1, dtype=np.float32)
_LOWER = [float(x) for x in _LIMS[:-1]]
_WIDTH = [float(b - a) for a, b in zip(_LIMS[:-1], _LIMS[1:])]

_SORT8 = [(0, 1), (2, 3), (4, 5), (6, 7),
          (0, 2), (1, 3), (4, 6), (5, 7),
          (1, 2), (5, 6),
          (0, 4), (1, 5), (2, 6), (3, 7),
          (2, 4), (3, 5),
          (1, 2), (3, 4), (5, 6)]


def _cmpx(v, i, j):
    a, b = v[i], v[j]
    if a is None and b is None:
        return
    if a is None:
        v[i], v[j] = b, None
        return
    if b is None:
        return
    v[i], v[j] = jnp.minimum(a, b), jnp.maximum(a, b)


def _sorted27(strat, surf):
    s = list(surf)
    for i, j in _SORT8:
        _cmpx(s, i, j)
    v = list(strat) + [None] * 5 + s[::-1]
    for d in (16, 8, 4, 2, 1):
        for i in range(32):
            if i % (2 * d) < d:
                _cmpx(v, i, i + d)
    out = v[:_S]
    assert all(x is not None for x in out)
    return out


def _body(depth_hbm, tbl_hbm, ib_hbm, ih_hbm, iw_hbm, sn_hbm, sf_hbm,
          oz_hbm, opc_hbm,
          ib_v, ih_v, iw_v, idx_a, idx_b, dep_v, tbl_v, sn_v, sf_v,
          oz_v, opc_v, sem):
    info = plsc.get_sparse_core_info()
    wid = lax.axis_index("s") * info.num_cores + lax.axis_index("c")
    base = wid * _RPW

    cps = [
        pltpu.async_copy(ib_hbm.at[pl.ds(base, _RPW)], ib_v, sem),
        pltpu.async_copy(ih_hbm.at[pl.ds(base, _RPW)], ih_v, sem),
        pltpu.async_copy(iw_hbm.at[pl.ds(base, _RPW)], iw_v, sem),
        pltpu.async_copy(sn_hbm.at[pl.ds(base * _NSTRAT, _RPW * _NSTRAT)],
                         sn_v, sem),
        pltpu.async_copy(sf_hbm.at[pl.ds(base * _NSURF, _RPW * _NSURF)],
                         sf_v, sem),
        pltpu.async_copy(tbl_hbm, tbl_v, sem),
    ]
    for cp in cps:
        cp.wait()

    def _flat_idx(idx_ref, half):
        @plsc.parallel_loop(0, 8, unroll=2)
        def b2(g2):
            off = half * 128 + g2 * 16
            b = ib_v[pl.ds(off, 16)]
            h = ih_v[pl.ds(off, 16)]
            w = iw_v[pl.ds(off, 16)]
            idx_ref[pl.ds(g2 * 16, 16)] = (b * _H + h) * _W + w

    _flat_idx(idx_a, 0)
    _flat_idx(idx_b, 1)
    ga = pltpu.async_copy(depth_hbm.at[idx_a], dep_v.at[pl.ds(0, 128)], sem)
    gb = pltpu.async_copy(depth_hbm.at[idx_b], dep_v.at[pl.ds(128, 128)], sem)
    ga.wait()
    gb.wait()

    iota = lax.iota(jnp.int32, 16)

    @plsc.parallel_loop(0, _NGRP, unroll=2)
    def grp(g):
        off = g * 16
        rows = off + iota
        b = ib_v[pl.ds(off, 16)]
        h = ih_v[pl.ds(off, 16)]
        w = iw_v[pl.ds(off, 16)]
        dx = (w.astype(jnp.float32) - _CX) * (1.0 / _FX)
        dy = (h.astype(jnp.float32) - _CY) * (1.0 / _FY)

        b16 = b * 16
        tk = [plsc.load_gather(tbl_v, [b16 + k]) for k in range(12)]
        dwx = tk[0] * dx + tk[1] * dy + tk[2]
        dwy = tk[4] * dx + tk[5] * dy + tk[6]
        dwz = tk[8] * dx + tk[9] * dy + tk[10]
        dirs = (dwx, dwy, dwz)
        orig = (tk[3], tk[7], tk[11])

        d = dep_v[pl.ds(off, 16)]
        maxd = d + jnp.sign(d + 1e-8) * _DIST_BEHIND
        rng = maxd - _MIN_DEPTH

        rsn = rows * _NSTRAT
        strat = []
        for s in range(_NSTRAT):
            nz = plsc.load_gather(sn_v, [rsn + s])
            t = _LOWER[s] + _WIDTH[s] * nz
            strat.append(_MIN_DEPTH + t * rng)
        rsf = rows * _NSURF
        surf = [d]
        for u in range(1, _NSURF):
            nz = plsc.load_gather(sf_v, [rsf + u])
            surf.append(d + nz * _SURF_OFF)

        zs = _sorted27(strat, surf)

        for s in range(_S):
            oz_v[s, pl.ds(off, 16)] = zs[s]
            for c in range(3):
                opc_v[c, s, pl.ds(off, 16)] = orig[c] + dirs[c] * zs[s]

    oa = pltpu.async_copy(oz_v, oz_hbm.at[:, pl.ds(base, _RPW)], sem)
    ob = pltpu.async_copy(opc_v, opc_hbm.at[:, :, pl.ds(base, _RPW)], sem)
    oa.wait()
    ob.wait()


_sens = functools.partial(
    pl.kernel,
    mesh=plsc.VectorSubcoreMesh(core_axis_name="c", subcore_axis_name="s"),
    out_type=[
        jax.ShapeDtypeStruct((_S, _N), jnp.float32),
        jax.ShapeDtypeStruct((3, _S, _N), jnp.float32),
    ],
    scratch_types=[
        pltpu.VMEM((_RPW,), jnp.int32),
        pltpu.VMEM((_RPW,), jnp.int32),
        pltpu.VMEM((_RPW,), jnp.int32),
        pltpu.VMEM((128,), jnp.int32),
        pltpu.VMEM((128,), jnp.int32),
        pltpu.VMEM((_RPW,), jnp.float32),
        pltpu.VMEM((_NF * 16,), jnp.float32),
        pltpu.VMEM((_RPW * _NSTRAT,), jnp.float32),
        pltpu.VMEM((_RPW * _NSURF,), jnp.float32),
        pltpu.VMEM((_S, _RPW), jnp.float32),
        pltpu.VMEM((3, _S, _RPW), jnp.float32),
        pltpu.SemaphoreType.DMA,
    ],
    compiler_params=pltpu.CompilerParams(
        needs_layout_passes=False,
        disable_bounds_checks=True,
        disable_semaphore_checks=True,
    ),
)(_body)


def kernel(depth_batch, T_WC_batch, indices_b, indices_h, indices_w,
           strat_noise, surf_noise):
    depth_flat = depth_batch.reshape(_NF * _H * _W)
    tbl = T_WC_batch.reshape(_NF * 16)
    zt, pct = _sens(depth_flat, tbl,
                    indices_b.astype(jnp.int32),
                    indices_h.astype(jnp.int32),
                    indices_w.astype(jnp.int32),
                    strat_noise.reshape(_N * _NSTRAT),
                    surf_noise.reshape(_N * _NSURF))
    return jnp.transpose(pct, (2, 1, 0)), zt.T


# speedup vs baseline: 1.0638x; 1.0273x over previous
import functools

import numpy as np
import jax
import jax.numpy as jnp
from jax import lax
from jax.experimental import pallas as pl
from jax.experimental.pallas import tpu as pltpu
from jax.experimental.pallas import tpu_sc as plsc

_H, _W = 480, 640
_FX, _FY, _CX, _CY = 600.0, 600.0, 319.5, 239.5
_NF = 8
_N = 8192
_NSTRAT = 19
_NSURF = 8
_S = _NSTRAT + _NSURF
_MIN_DEPTH = 0.07
_DIST_BEHIND = 0.1
_SURF_OFF = 0.05

_RPW = _N // 32
_NGRP = _RPW // 16

_LIMS = np.linspace(0.0, 1.0, _NSTRAT + 1, dtype=np.float32)
_LOWER = [float(x) for x in _LIMS[:-1]]
_WIDTH = [float(b - a) for a, b in zip(_LIMS[:-1], _LIMS[1:])]

_SORT8 = [(0, 1), (2, 3), (4, 5), (6, 7),
          (0, 2), (1, 3), (4, 6), (5, 7),
          (1, 2), (5, 6),
          (0, 4), (1, 5), (2, 6), (3, 7),
          (2, 4), (3, 5),
          (1, 2), (3, 4), (5, 6)]


def _cmpx(v, i, j):
    a, b = v[i], v[j]
    if a is None and b is None:
        return
    if a is None:
        v[i], v[j] = b, None
        return
    if b is None:
        return
    v[i], v[j] = jnp.minimum(a, b), jnp.maximum(a, b)


def _sorted27(strat, surf):
    s = list(surf)
    for i, j in _SORT8:
        _cmpx(s, i, j)
    v = list(strat) + [None] * 5 + s[::-1]
    for d in (16, 8, 4, 2, 1):
        for i in range(32):
            if i % (2 * d) < d:
                _cmpx(v, i, i + d)
    out = v[:_S]
    assert all(x is not None for x in out)
    return out


def _body(depth_hbm, tbl_hbm, ib_hbm, ih_hbm, iw_hbm, sn_hbm, sf_hbm,
          oz_hbm, opc_hbm,
          ib_v, ih_v, iw_v, idx_a, idx_b, dep_v, tbl_v, sn_v, sf_v,
          oz_v, opc_v, sem, sem2):
    info = plsc.get_sparse_core_info()
    wid = lax.axis_index("s") * info.num_cores + lax.axis_index("c")
    base = wid * _RPW

    cps = [
        pltpu.async_copy(ib_hbm.at[pl.ds(base, _RPW)], ib_v, sem),
        pltpu.async_copy(ih_hbm.at[pl.ds(base, _RPW)], ih_v, sem),
        pltpu.async_copy(iw_hbm.at[pl.ds(base, _RPW)], iw_v, sem),
    ]
    cps2 = [
        pltpu.async_copy(sn_hbm.at[pl.ds(base * _NSTRAT, _RPW * _NSTRAT)],
                         sn_v, sem2),
        pltpu.async_copy(sf_hbm.at[pl.ds(base * _NSURF, _RPW * _NSURF)],
                         sf_v, sem2),
        pltpu.async_copy(tbl_hbm, tbl_v, sem2),
    ]
    for cp in cps:
        cp.wait()

    def _flat_idx(idx_ref, half):
        @plsc.parallel_loop(0, 8, unroll=2)
        def b2(g2):
            off = half * 128 + g2 * 16
            b = ib_v[pl.ds(off, 16)]
            h = ih_v[pl.ds(off, 16)]
            w = iw_v[pl.ds(off, 16)]
            idx_ref[pl.ds(g2 * 16, 16)] = (b * _H + h) * _W + w

    _flat_idx(idx_a, 0)
    _flat_idx(idx_b, 1)
    ga = pltpu.async_copy(depth_hbm.at[idx_a], dep_v.at[pl.ds(0, 128)], sem)
    gb = pltpu.async_copy(depth_hbm.at[idx_b], dep_v.at[pl.ds(128, 128)], sem)
    ga.wait()
    gb.wait()
    for cp in cps2:
        cp.wait()

    iota = lax.iota(jnp.int32, 16)

    def grp_range(glo, ghi):
        @plsc.parallel_loop(glo, ghi, unroll=2)
        def grp(g):
            _do_group(g)

    def _do_group(g):
        off = g * 16
        rows = off + iota
        b = ib_v[pl.ds(off, 16)]
        h = ih_v[pl.ds(off, 16)]
        w = iw_v[pl.ds(off, 16)]
        dx = (w.astype(jnp.float32) - _CX) * (1.0 / _FX)
        dy = (h.astype(jnp.float32) - _CY) * (1.0 / _FY)

        b16 = b * 16
        tk = [plsc.load_gather(tbl_v, [b16 + k]) for k in range(12)]
        dwx = tk[0] * dx + tk[1] * dy + tk[2]
        dwy = tk[4] * dx + tk[5] * dy + tk[6]
        dwz = tk[8] * dx + tk[9] * dy + tk[10]
        dirs = (dwx, dwy, dwz)
        orig = (tk[3], tk[7], tk[11])

        d = dep_v[pl.ds(off, 16)]
        maxd = d + jnp.sign(d + 1e-8) * _DIST_BEHIND
        rng = maxd - _MIN_DEPTH

        rsn = rows * _NSTRAT
        strat = []
        for s in range(_NSTRAT):
            nz = plsc.load_gather(sn_v, [rsn + s])
            t = _LOWER[s] + _WIDTH[s] * nz
            strat.append(_MIN_DEPTH + t * rng)
        rsf = rows * _NSURF
        surf = [d]
        for u in range(1, _NSURF):
            nz = plsc.load_gather(sf_v, [rsf + u])
            surf.append(d + nz * _SURF_OFF)

        zs = _sorted27(strat, surf)

        for s in range(_S):
            oz_v[s, pl.ds(off, 16)] = zs[s]
            for c in range(3):
                opc_v[c, s, pl.ds(off, 16)] = orig[c] + dirs[c] * zs[s]

    half = _RPW // 2
    grp_range(0, _NGRP // 2)
    o1 = pltpu.async_copy(oz_v.at[:, pl.ds(0, half)],
                          oz_hbm.at[:, pl.ds(base, half)], sem)
    o2 = pltpu.async_copy(opc_v.at[:, :, pl.ds(0, half)],
                          opc_hbm.at[:, :, pl.ds(base, half)], sem)
    grp_range(_NGRP // 2, _NGRP)
    o3 = pltpu.async_copy(oz_v.at[:, pl.ds(half, half)],
                          oz_hbm.at[:, pl.ds(base + half, half)], sem)
    o4 = pltpu.async_copy(opc_v.at[:, :, pl.ds(half, half)],
                          opc_hbm.at[:, :, pl.ds(base + half, half)], sem)
    o1.wait()
    o2.wait()
    o3.wait()
    o4.wait()


_sens = functools.partial(
    pl.kernel,
    mesh=plsc.VectorSubcoreMesh(core_axis_name="c", subcore_axis_name="s"),
    out_type=[
        jax.ShapeDtypeStruct((_S, _N), jnp.float32),
        jax.ShapeDtypeStruct((3, _S, _N), jnp.float32),
    ],
    scratch_types=[
        pltpu.VMEM((_RPW,), jnp.int32),
        pltpu.VMEM((_RPW,), jnp.int32),
        pltpu.VMEM((_RPW,), jnp.int32),
        pltpu.VMEM((128,), jnp.int32),
        pltpu.VMEM((128,), jnp.int32),
        pltpu.VMEM((_RPW,), jnp.float32),
        pltpu.VMEM((_NF * 16,), jnp.float32),
        pltpu.VMEM((_RPW * _NSTRAT,), jnp.float32),
        pltpu.VMEM((_RPW * _NSURF,), jnp.float32),
        pltpu.VMEM((_S, _RPW), jnp.float32),
        pltpu.VMEM((3, _S, _RPW), jnp.float32),
        pltpu.SemaphoreType.DMA,
        pltpu.SemaphoreType.DMA,
    ],
    compiler_params=pltpu.CompilerParams(
        needs_layout_passes=False,
        disable_bounds_checks=True,
        disable_semaphore_checks=True,
    ),
)(_body)


def kernel(depth_batch, T_WC_batch, indices_b, indices_h, indices_w,
           strat_noise, surf_noise):
    depth_flat = depth_batch.reshape(_NF * _H * _W)
    tbl = T_WC_batch.reshape(_NF * 16)
    zt, pct = _sens(depth_flat, tbl,
                    indices_b.astype(jnp.int32),
                    indices_h.astype(jnp.int32),
                    indices_w.astype(jnp.int32),
                    strat_noise.reshape(_N * _NSTRAT),
                    surf_noise.reshape(_N * _NSURF))
    return jnp.transpose(pct, (2, 1, 0)), zt.T
